# Initial kernel scaffold; baseline (speedup 1.0000x reference)
import jax
import jax.numpy as jnp
from jax.experimental import pallas as pl


def kernel(inputs, W):
    raise NotImplementedError("write your pallas kernel here")


# speedup vs baseline: 1.2131x; 1.2131x over previous
import jax
import jax.numpy as jnp
from jax.experimental import pallas as pl
from jax.experimental.pallas import tpu as pltpu

_NUM_EMBEDDINGS = 1024
_EMBEDDING_DIM = 64
_COMMITMENT_COST = 0.25
_SPATIAL = 32 * 32
_BATCH = 16
_N_TOTAL = _BATCH * _SPATIAL * _EMBEDDING_DIM


def _vq_body(x_ref, w_ref, q_ref, idx_ref, loss_ref):
    x = x_ref[0]
    w = w_ref[...]

    xsq = jnp.sum(x * x, axis=0, keepdims=True)
    wsq = jnp.sum(w * w, axis=1, keepdims=True)
    xw = jax.lax.dot_general(
        w, x, (((1,), (0,)), ((), ())),
        preferred_element_type=jnp.float32)
    dist = (xsq + wsq) - 2.0 * xw

    min_d = jnp.min(dist, axis=0, keepdims=True)
    code_iota = jax.lax.broadcasted_iota(
        jnp.int32, (_NUM_EMBEDDINGS, _SPATIAL), 0)
    idx = jnp.min(
        jnp.where(dist == min_d, code_iota, _NUM_EMBEDDINGS), axis=0)

    onehot = (code_iota == idx[None, :]).astype(jnp.float32)
    q = jax.lax.dot_general(
        w, onehot, (((0,), (0,)), ((), ())),
        precision=jax.lax.Precision.HIGHEST,
        preferred_element_type=jnp.float32)

    q_ref[0] = x + (q - x)
    idx_ref[0, 0, :] = idx

    diff = q - x
    part = jnp.sum(diff * diff, axis=(0, 1), keepdims=True)

    @pl.when(pl.program_id(0) == 0)
    def _():
        loss_ref[...] = jnp.zeros((1, 1), jnp.float32)
    total = loss_ref[...] + part
    scaled = _COMMITMENT_COST * (total / _N_TOTAL) * 10.0
    loss_ref[...] = jnp.where(
        pl.program_id(0) == _BATCH - 1, scaled, total)


def kernel(inputs, W):
    x = inputs.astype(jnp.float32).reshape(_BATCH, _EMBEDDING_DIM, _SPATIAL)
    q, idx, loss = pl.pallas_call(
        _vq_body,
        grid=(_BATCH,),
        in_specs=[
            pl.BlockSpec((1, _EMBEDDING_DIM, _SPATIAL), lambda i: (i, 0, 0)),
            pl.BlockSpec((_NUM_EMBEDDINGS, _EMBEDDING_DIM), lambda i: (0, 0)),
        ],
        out_specs=[
            pl.BlockSpec((1, _EMBEDDING_DIM, _SPATIAL), lambda i: (i, 0, 0)),
            pl.BlockSpec((1, 1, _SPATIAL), lambda i: (i, 0, 0)),
            pl.BlockSpec((1, 1), lambda i: (0, 0)),
        ],
        out_shape=[
            jax.ShapeDtypeStruct((_BATCH, _EMBEDDING_DIM, _SPATIAL), jnp.float32),
            jax.ShapeDtypeStruct((_BATCH, 1, _SPATIAL), jnp.int32),
            jax.ShapeDtypeStruct((1, 1), jnp.float32),
        ],
        compiler_params=pltpu.CompilerParams(
            dimension_semantics=("arbitrary",),
        ),
    )(x, W)

    quantized_st = q.reshape(_BATCH, _EMBEDDING_DIM, 32, 32)
    encoding_indices = idx.reshape(_BATCH, 32, 32)
    encodings_sum = jnp.zeros(256, dtype=jnp.float32)
    return (quantized_st, loss[0, 0], encoding_indices, encodings_sum, W)
